# baseline (speedup 1.0000x reference)
import functools

import jax
import jax.numpy as jnp
from jax import lax
from jax.experimental import pallas as pl
from jax.experimental.pallas import tpu as pltpu
from jax.experimental.pallas import tpu_sc as plsc

N_NODES = 10000
N_EDGES = 320000
D_IN = 128
D_HID = 128
D_OUT = 64

NPAD = 10240
EPAD = 327680
CH = 128
NCHUNK = EPAD // CH
CPT = NCHUNK // 32
RPT = NPAD // 16
BLK = 1024
GRID = NPAD // BLK

_MESH = dict(core_axis_name="c", subcore_axis_name="s", num_cores=2,
             num_subcores=16)


def _make_sc_agg(d, tc_tiling, do_scatter=True):
    mesh = plsc.VectorSubcoreMesh(**_MESH)
    params = None if tc_tiling else pltpu.CompilerParams(
        use_tc_tiling_on_sc=False)

    @functools.partial(
        pl.kernel,
        out_type=jax.ShapeDtypeStruct((2, NPAD, d), jnp.float32),
        mesh=mesh,
        compiler_params=params,
        scratch_types=[
            pltpu.VMEM_SHARED((NPAD, d), jnp.float32),
            pltpu.VMEM((4, 2, CH), jnp.int32),
            pltpu.VMEM((2, CH, d), jnp.float32),
            pltpu.VMEM((16, d), jnp.float32),
            pltpu.SemaphoreType.DMA,
            pltpu.SemaphoreType.DMA,
            pltpu.SemaphoreType.DMA,
            pltpu.SemaphoreType.DMA,
            pltpu.SemaphoreType.DMA,
            pltpu.SemaphoreType.DMA,
        ],
    )
    def k(inter_hbm, g_hbm, out_hbm, acc, idxb, rows, zb,
          si0, si1, si2, si3, sg0, sg1):
        c = lax.axis_index("c")
        s = lax.axis_index("s")
        base = (c * 16 + s) * CPT
        last = base + CPT - 1
        si = (si0, si1, si2, si3)
        sg = (sg0, sg1)

        def idx_load(j, ib, sem):
            pltpu.async_copy(inter_hbm.at[j], idxb.at[ib], sem)

        def idx_wait(j, ib, sem):
            pltpu.make_async_copy(inter_hbm.at[j], idxb.at[ib], sem).wait()

        def gather(j_unused, ib, rb, sem):
            pltpu.async_copy(g_hbm.at[idxb.at[ib, 0]], rows.at[rb], sem)

        def gather_wait(ib, rb, sem):
            pltpu.make_async_copy(g_hbm.at[idxb.at[ib, 0]], rows.at[rb],
                                  sem).wait()

        def scatter(ib, rb):
            pltpu.sync_copy(rows.at[rb], acc.at[idxb.at[ib, 1]], add=True)

        pltpu.sync_copy(inter_hbm.at[base], idxb.at[0])
        idx_load(base + 1, 1, si1)
        idx_load(base + 2, 2, si2)
        idx_load(base + 3, 3, si3)
        gather(base, 0, 0, sg0)
        idx_wait(base + 1, 1, si1)
        gather(base + 1, 1, 1, sg1)

        zv = jnp.zeros((16,), jnp.float32)
        for r in range(16):
            for q in range(d // 16):
                zb[r, pl.ds(q * 16, 16)] = zv

        def zbody(i, _):
            pltpu.sync_copy(zb, acc.at[pl.ds(s * RPT + i * 16, 16)])
            return ()

        lax.fori_loop(0, RPT // 16, zbody, ())
        plsc.subcore_barrier()

        def body(i, _):
            j = base + i * 4
            for t in range(4):
                ib = t
                rb = t % 2
                gather_wait(ib, rb, sg[rb])
                scatter(ib, rb)
                idx_load(jnp.minimum(j + t + 4, last), ib, si[ib])
                ibn = (t + 2) % 4
                idx_wait(jnp.minimum(j + t + 2, last), ibn, si[ibn])
                gather(j + t + 2, ibn, rb, sg[rb])
            return ()

        lax.fori_loop(0, CPT // 4, body, ())
        gather_wait(0, 0, sg0)
        gather_wait(1, 1, sg1)
        idx_wait(last, 2, si2)
        idx_wait(last, 3, si3)
        plsc.subcore_barrier()

        pltpu.sync_copy(acc.at[pl.ds(s * RPT, RPT)],
                        out_hbm.at[c, pl.ds(s * RPT, RPT)])

    return k


_sc_agg128 = _make_sc_agg(D_HID, tc_tiling=True)
_sc_agg64 = _make_sc_agg(D_HID, tc_tiling=True)


def _make_sc_deg():
    mesh = plsc.VectorSubcoreMesh(**_MESH)

    @functools.partial(
        pl.kernel,
        out_type=jax.ShapeDtypeStruct((2, NPAD, 16), jnp.float32),
        mesh=mesh,
        scratch_types=[
            pltpu.VMEM_SHARED((NPAD, 16), jnp.float32),
            pltpu.VMEM((CPT, CH), jnp.int32),
            pltpu.VMEM((CH, 16), jnp.float32),
            pltpu.SemaphoreType.DMA,
        ],
    )
    def k(dst_hbm, ones_hbm, out_hbm, acc, dstb, onesb, sd):
        c = lax.axis_index("c")
        s = lax.axis_index("s")
        w = c * 16 + s

        zv = jnp.zeros((16,), jnp.float32)
        for r in range(16):
            onesb[r, :] = zv

        def zbody(i, _):
            pltpu.sync_copy(onesb.at[pl.ds(0, 16)],
                            acc.at[pl.ds(s * RPT + i * 16, 16)])
            return ()

        lax.fori_loop(0, RPT // 16, zbody, ())
        pltpu.sync_copy(ones_hbm, onesb)
        pltpu.sync_copy(dst_hbm.at[pl.ds(w * CPT, CPT)], dstb)
        plsc.subcore_barrier()

        def body(j, _):
            pltpu.async_copy(onesb, acc.at[dstb.at[j]], sd, add=True)
            return ()

        lax.fori_loop(0, CPT, body, ())

        def drain(j, _):
            pltpu.make_async_copy(onesb, acc.at[dstb.at[j]], sd).wait()
            return ()

        lax.fori_loop(0, CPT, drain, ())
        plsc.subcore_barrier()

        pltpu.sync_copy(acc.at[pl.ds(s * RPT, RPT)],
                        out_hbm.at[c, pl.ds(s * RPT, RPT)])

    return k


_sc_deg = _make_sc_deg()


def _dinv_block(degp_ref):
    deg = degp_ref[0, :, 0:1] + degp_ref[1, :, 0:1] + 1.0
    return lax.rsqrt(deg)


def _tc_h1_body(x_ref, w1_ref, h1_ref):
    h1_ref[...] = jnp.dot(x_ref[...], w1_ref[...],
                          preferred_element_type=jnp.float32)


def _tc_scale_body(h1_ref, degp_ref, g1_ref):
    g1_ref[...] = h1_ref[...] * _dinv_block(degp_ref)


def _tc_mid_body(p_ref, g1_ref, degp_ref, b1_ref, w2_ref, g2_ref):
    dinv = _dinv_block(degp_ref)
    tot = p_ref[0] + p_ref[1] + g1_ref[...]
    h = jnp.maximum(tot * dinv + b1_ref[...], 0.0)
    g2 = jnp.dot(h, w2_ref[...], preferred_element_type=jnp.float32) * dinv
    g2_ref[...] = jnp.pad(g2, ((0, 0), (0, D_HID - D_OUT)))


def _tc_out_body(q_ref, g2_ref, degp_ref, b2_ref, out_ref):
    dinv = _dinv_block(degp_ref)
    tot = q_ref[0] + q_ref[1] + g2_ref[...]
    out_ref[...] = jnp.maximum(tot * dinv + b2_ref[...], 0.0)


def _node_spec(d):
    return pl.BlockSpec((BLK, d), lambda i: (i, 0))


def _pair_spec(d):
    return pl.BlockSpec((2, BLK, d), lambda i: (0, i, 0))


def _full_spec(r, c):
    return pl.BlockSpec((r, c), lambda i: (0, 0))


_deg_spec = _pair_spec(16)

_tc_h1 = pl.pallas_call(
    _tc_h1_body,
    grid=(GRID,),
    in_specs=[_node_spec(D_IN), _full_spec(D_IN, D_HID)],
    out_specs=_node_spec(D_HID),
    out_shape=jax.ShapeDtypeStruct((NPAD, D_HID), jnp.float32),
)

_tc_scale = pl.pallas_call(
    _tc_scale_body,
    grid=(GRID,),
    in_specs=[_node_spec(D_HID), _deg_spec],
    out_specs=_node_spec(D_HID),
    out_shape=jax.ShapeDtypeStruct((NPAD, D_HID), jnp.float32),
)

_tc_mid = pl.pallas_call(
    _tc_mid_body,
    grid=(GRID,),
    in_specs=[_pair_spec(D_HID), _node_spec(D_HID), _deg_spec,
              _full_spec(1, D_HID), _full_spec(D_HID, D_OUT)],
    out_specs=_node_spec(D_HID),
    out_shape=jax.ShapeDtypeStruct((NPAD, D_HID), jnp.float32),
)

_tc_out = pl.pallas_call(
    _tc_out_body,
    grid=(GRID,),
    in_specs=[_pair_spec(D_HID), _node_spec(D_HID), _deg_spec,
              _full_spec(1, D_HID)],
    out_specs=_node_spec(D_HID),
    out_shape=jax.ShapeDtypeStruct((NPAD, D_HID), jnp.float32),
)


@jax.jit
def kernel(x, edge_index, W1, b1, W2, b2):
    xp = jnp.pad(x, ((0, NPAD - N_NODES), (0, 0)))
    src = edge_index[0].astype(jnp.int32)
    dst = edge_index[1].astype(jnp.int32)
    pad = N_NODES + jnp.arange(EPAD - N_EDGES, dtype=jnp.int32) % (
        NPAD - N_NODES)
    src2d = jnp.concatenate([src, pad]).reshape(NCHUNK, CH)
    dst2d = jnp.concatenate([dst, pad]).reshape(NCHUNK, CH)
    inter = jnp.stack([src2d, dst2d], axis=1)

    h1 = _tc_h1(xp, W1)
    degp = _sc_deg(dst2d, jnp.ones((CH, 16), jnp.float32))
    g1 = _tc_scale(h1, degp)
    p = _sc_agg128(inter, g1)
    g2 = _tc_mid(p, g1, degp, b1.reshape(1, D_HID), W2)
    q = _sc_agg64(inter, g2)
    b2p = jnp.pad(b2, (0, D_HID - D_OUT)).reshape(1, D_HID)
    out = _tc_out(q, g2, degp, b2p)
    return out[:N_NODES, :D_OUT]


# speedup vs baseline: 1.0931x; 1.0931x over previous
import functools

import jax
import jax.numpy as jnp
from jax import lax
from jax.experimental import pallas as pl
from jax.experimental.pallas import tpu as pltpu
from jax.experimental.pallas import tpu_sc as plsc

N_NODES = 10000
N_EDGES = 320000
D_IN = 128
D_HID = 128
D_OUT = 64

NPAD = 10240
EPAD = 327680
CH = 128
NCHUNK = EPAD // CH
CPT = NCHUNK // 32
CHA = 80
NCHUNKA = EPAD // CHA
CPTA = NCHUNKA // 32
RPT = NPAD // 16
BLK = 1024
GRID = NPAD // BLK

_MESH = dict(core_axis_name="c", subcore_axis_name="s", num_cores=2,
             num_subcores=16)


def _make_sc_agg(d):
    mesh = plsc.VectorSubcoreMesh(**_MESH)

    @functools.partial(
        pl.kernel,
        out_type=jax.ShapeDtypeStruct((2, NPAD, d), jnp.float32),
        mesh=mesh,
        scratch_types=[
            pltpu.VMEM_SHARED((NPAD, d), jnp.float32),
            pltpu.VMEM((8, 2, CHA), jnp.int32),
            pltpu.VMEM((4, CHA, d), jnp.float32),
            pltpu.VMEM((16, d), jnp.float32),
            [pltpu.SemaphoreType.DMA] * 8,
            [pltpu.SemaphoreType.DMA] * 4,
        ],
    )
    def k(inter_hbm, g_hbm, out_hbm, acc, idxb, rows, zb, si, sg):
        c = lax.axis_index("c")
        s = lax.axis_index("s")
        base = (c * 16 + s) * CPTA
        last = base + CPTA - 1

        def idx_load(j, ib):
            pltpu.async_copy(inter_hbm.at[j], idxb.at[ib], si[ib])

        def idx_wait(j, ib):
            pltpu.make_async_copy(inter_hbm.at[j], idxb.at[ib], si[ib]).wait()

        def gather(ib, rb):
            pltpu.async_copy(g_hbm.at[idxb.at[ib, 0]], rows.at[rb], sg[rb])

        def gather_wait(ib, rb):
            pltpu.make_async_copy(g_hbm.at[idxb.at[ib, 0]], rows.at[rb],
                                  sg[rb]).wait()

        def scatter(ib, rb):
            pltpu.sync_copy(rows.at[rb], acc.at[idxb.at[ib, 1]], add=True)

        pltpu.sync_copy(inter_hbm.at[base], idxb.at[0])
        for t in range(1, 8):
            idx_load(base + t, t)
        gather(0, 0)
        for t in range(1, 4):
            idx_wait(base + t, t)
            gather(t, t)

        zv = jnp.zeros((16,), jnp.float32)
        for r in range(16):
            for q in range(d // 16):
                zb[r, pl.ds(q * 16, 16)] = zv

        def zbody(i, _):
            pltpu.sync_copy(zb, acc.at[pl.ds(s * RPT + i * 16, 16)])
            return ()

        lax.fori_loop(0, RPT // 16, zbody, ())
        plsc.subcore_barrier()

        def body(i, _):
            j = base + i * 8
            for t in range(8):
                ib = t
                rb = t % 4
                gather_wait(ib, rb)
                scatter(ib, rb)
                idx_load(jnp.minimum(j + t + 8, last), ib)
                ibn = (t + 4) % 8
                idx_wait(jnp.minimum(j + t + 4, last), ibn)
                gather(ibn, rb)
            return ()

        lax.fori_loop(0, CPTA // 8, body, ())
        for rb in range(4):
            gather_wait(rb, rb)
        for ib in range(4, 8):
            idx_wait(last, ib)
        plsc.subcore_barrier()

        pltpu.sync_copy(acc.at[pl.ds(s * RPT, RPT)],
                        out_hbm.at[c, pl.ds(s * RPT, RPT)])

    return k


_sc_agg128 = _make_sc_agg(D_HID)
_sc_agg64 = _make_sc_agg(D_HID)


def _make_sc_deg():
    mesh = plsc.VectorSubcoreMesh(**_MESH)

    @functools.partial(
        pl.kernel,
        out_type=jax.ShapeDtypeStruct((2, NPAD, 16), jnp.float32),
        mesh=mesh,
        scratch_types=[
            pltpu.VMEM_SHARED((NPAD, 16), jnp.float32),
            pltpu.VMEM((CPT, CH), jnp.int32),
            pltpu.VMEM((CH, 16), jnp.float32),
            pltpu.SemaphoreType.DMA,
        ],
    )
    def k(dst_hbm, ones_hbm, out_hbm, acc, dstb, onesb, sd):
        c = lax.axis_index("c")
        s = lax.axis_index("s")
        w = c * 16 + s

        zv = jnp.zeros((16,), jnp.float32)
        for r in range(16):
            onesb[r, :] = zv

        def zbody(i, _):
            pltpu.sync_copy(onesb.at[pl.ds(0, 16)],
                            acc.at[pl.ds(s * RPT + i * 16, 16)])
            return ()

        lax.fori_loop(0, RPT // 16, zbody, ())
        pltpu.sync_copy(ones_hbm, onesb)
        pltpu.sync_copy(dst_hbm.at[pl.ds(w * CPT, CPT)], dstb)
        plsc.subcore_barrier()

        def body(j, _):
            pltpu.async_copy(onesb, acc.at[dstb.at[j]], sd, add=True)
            return ()

        lax.fori_loop(0, CPT, body, ())

        def drain(j, _):
            pltpu.make_async_copy(onesb, acc.at[dstb.at[j]], sd).wait()
            return ()

        lax.fori_loop(0, CPT, drain, ())
        plsc.subcore_barrier()

        pltpu.sync_copy(acc.at[pl.ds(s * RPT, RPT)],
                        out_hbm.at[c, pl.ds(s * RPT, RPT)])

    return k


_sc_deg = _make_sc_deg()


def _dinv_block(degp_ref):
    deg = degp_ref[0, :, 0:1] + degp_ref[1, :, 0:1] + 1.0
    return lax.rsqrt(deg)


def _tc_g1_body(x_ref, w1_ref, degp_ref, g1_ref):
    dinv = _dinv_block(degp_ref)
    g1_ref[...] = jnp.dot(x_ref[...], w1_ref[...],
                          preferred_element_type=jnp.float32) * dinv


def _tc_mid_body(p_ref, g1_ref, degp_ref, b1_ref, w2_ref, g2_ref):
    dinv = _dinv_block(degp_ref)
    tot = p_ref[0] + p_ref[1] + g1_ref[...]
    h = jnp.maximum(tot * dinv + b1_ref[...], 0.0)
    g2 = jnp.dot(h, w2_ref[...], preferred_element_type=jnp.float32) * dinv
    g2_ref[...] = jnp.pad(g2, ((0, 0), (0, D_HID - D_OUT)))


def _tc_out_body(q_ref, g2_ref, degp_ref, b2_ref, out_ref):
    dinv = _dinv_block(degp_ref)
    tot = q_ref[0] + q_ref[1] + g2_ref[...]
    out_ref[...] = jnp.maximum(tot * dinv + b2_ref[...], 0.0)


def _node_spec(d):
    return pl.BlockSpec((BLK, d), lambda i: (i, 0))


def _pair_spec(d):
    return pl.BlockSpec((2, BLK, d), lambda i: (0, i, 0))


def _full_spec(r, c):
    return pl.BlockSpec((r, c), lambda i: (0, 0))


_deg_spec = _pair_spec(16)

_tc_g1 = pl.pallas_call(
    _tc_g1_body,
    grid=(GRID,),
    in_specs=[_node_spec(D_IN), _full_spec(D_IN, D_HID), _deg_spec],
    out_specs=_node_spec(D_HID),
    out_shape=jax.ShapeDtypeStruct((NPAD, D_HID), jnp.float32),
)

_tc_mid = pl.pallas_call(
    _tc_mid_body,
    grid=(GRID,),
    in_specs=[_pair_spec(D_HID), _node_spec(D_HID), _deg_spec,
              _full_spec(1, D_HID), _full_spec(D_HID, D_OUT)],
    out_specs=_node_spec(D_HID),
    out_shape=jax.ShapeDtypeStruct((NPAD, D_HID), jnp.float32),
)

_tc_out = pl.pallas_call(
    _tc_out_body,
    grid=(GRID,),
    in_specs=[_pair_spec(D_HID), _node_spec(D_HID), _deg_spec,
              _full_spec(1, D_HID)],
    out_specs=_node_spec(D_HID),
    out_shape=jax.ShapeDtypeStruct((NPAD, D_HID), jnp.float32),
)


@jax.jit
def kernel(x, edge_index, W1, b1, W2, b2):
    xp = jnp.pad(x, ((0, NPAD - N_NODES), (0, 0)))
    src = edge_index[0].astype(jnp.int32)
    dst = edge_index[1].astype(jnp.int32)
    pad = N_NODES + jnp.arange(EPAD - N_EDGES, dtype=jnp.int32) % (
        NPAD - N_NODES)
    srcp = jnp.concatenate([src, pad])
    dstp = jnp.concatenate([dst, pad])
    dst2d = dstp.reshape(NCHUNK, CH)
    inter = jnp.stack([srcp.reshape(NCHUNKA, CHA),
                       dstp.reshape(NCHUNKA, CHA)], axis=1)

    degp = _sc_deg(dst2d, jnp.ones((CH, 16), jnp.float32))
    g1 = _tc_g1(xp, W1, degp)
    p = _sc_agg128(inter, g1)
    g2 = _tc_mid(p, g1, degp, b1.reshape(1, D_HID), W2)
    q = _sc_agg64(inter, g2)
    b2p = jnp.pad(b2, (0, D_HID - D_OUT)).reshape(1, D_HID)
    out = _tc_out(q, g2, degp, b2p)
    return out[:N_NODES, :D_OUT]
